# baseline (speedup 1.0000x reference)
import functools

import jax
import jax.numpy as jnp
from jax.experimental import pallas as pl
from jax.experimental.pallas import tpu as pltpu

TOPK = 64
_CHUNK = 128
_QDEPTH = 8


def _matmul_body(xh_ref, xl_ref, wh_ref, wl_ref, b_ref, out_ref):
    dn = (((1,), (1,)), ((), ()))
    xh = xh_ref[...]
    wh = wh_ref[...]
    out_ref[...] = (
        jax.lax.dot_general(xh, wh, dn, preferred_element_type=jnp.float32)
        + (jax.lax.dot_general(xh, wl_ref[...], dn,
                               preferred_element_type=jnp.float32)
           + jax.lax.dot_general(xl_ref[...], wh, dn,
                                 preferred_element_type=jnp.float32))
        + b_ref[...])


def _encode_matmul(x_hi, x_lo, W_hi, W_lo, b_enc):
    b, i_dim = x_hi.shape
    d = W_hi.shape[0]
    r = min(1024, b)
    n_tile = min(256, d)
    grid = (b // r, d // n_tile)
    return pl.pallas_call(
        _matmul_body,
        grid=grid,
        in_specs=[
            pl.BlockSpec((r, i_dim), lambda i, j: (i, 0)),
            pl.BlockSpec((r, i_dim), lambda i, j: (i, 0)),
            pl.BlockSpec((n_tile, i_dim), lambda i, j: (j, 0)),
            pl.BlockSpec((n_tile, i_dim), lambda i, j: (j, 0)),
            pl.BlockSpec((1, n_tile), lambda i, j: (0, j)),
        ],
        out_specs=pl.BlockSpec((r, n_tile), lambda i, j: (i, j)),
        out_shape=jax.ShapeDtypeStruct((b, d), jnp.float32),
        compiler_params=pltpu.CompilerParams(
            dimension_semantics=("parallel", "arbitrary")),
    )(x_hi, x_lo, W_hi, W_lo, b_enc.reshape(1, d))


def _topk_body(code_ref, sc_out, idx_out, val_out, qav_ref, qsv_ref, qix_ref,
               *, k, r, d):
    w = _CHUNK
    nc = d // w
    q_depth = _QDEPTH
    code = code_ref[...]
    c3 = code.reshape(r, nc, w)
    iota_w = jax.lax.broadcasted_iota(jnp.int32, (r, nc, w), 2)

    a3 = jnp.abs(c3)
    for q in range(q_depth):
        m = jnp.max(a3, axis=2)
        sel = jnp.min(jnp.where(a3 == m[:, :, None], iota_w, w), axis=2)
        is_sel = iota_w == sel[:, :, None]
        qav_ref[:, q, :] = m
        qsv_ref[:, q, :] = jnp.sum(jnp.where(is_sel, c3, 0.0), axis=2)
        qix_ref[:, q, :] = sel
        a3 = jnp.where(is_sel, -1.0, a3)

    iota_q = jax.lax.broadcasted_iota(jnp.int32, (r, q_depth, nc), 1)
    iota_nc = jax.lax.broadcasted_iota(jnp.int32, (r, nc), 1)
    iota_k = jax.lax.broadcasted_iota(jnp.int32, (r, k), 1)
    qav = qav_ref[...]
    qsv = qsv_ref[...]
    qix = qix_ref[...]

    def body(t, carry):
        head, ptr, vals, idxs, _ = carry
        m = jnp.max(head, axis=1, keepdims=True)
        c = jnp.min(jnp.where(head == m, iota_nc, nc), axis=1, keepdims=True)
        onehot = iota_nc == c
        mask3 = onehot[:, None, :] & (iota_q == ptr[:, None, :])
        v = jnp.sum(jnp.sum(jnp.where(mask3, qsv, 0.0), axis=1), axis=1,
                    keepdims=True)
        wi = jnp.sum(jnp.sum(jnp.where(mask3, qix, 0), axis=1), axis=1,
                     keepdims=True)
        gidx = c * w + wi
        ptr = ptr + onehot.astype(jnp.int32)
        head_new = jnp.max(jnp.where(iota_q == ptr[:, None, :], qav, -1.0),
                           axis=1)
        head = jnp.where(onehot, head_new, head)
        vals = jnp.where(iota_k == t, v, vals)
        idxs = jnp.where(iota_k == t, gidx, idxs)
        return head, ptr, vals, idxs, m

    init = (qav_ref[:, 0, :], jnp.zeros((r, nc), jnp.int32),
            jnp.zeros((r, k), jnp.float32), jnp.zeros((r, k), jnp.int32),
            jnp.zeros((r, 1), jnp.float32))
    _, _, vals, idxs, th = jax.lax.fori_loop(0, k, body, init)
    val_out[...] = vals
    idx_out[...] = idxs
    sc_out[...] = jnp.where(jnp.abs(code) >= th, code, 0.0)


def _topk(dense_code, k):
    b, d = dense_code.shape
    r = min(64, b)
    grid = (b // r,)
    body = functools.partial(_topk_body, k=k, r=r, d=d)
    return pl.pallas_call(
        body,
        grid=grid,
        in_specs=[pl.BlockSpec((r, d), lambda i: (i, 0))],
        out_specs=[
            pl.BlockSpec((r, d), lambda i: (i, 0)),
            pl.BlockSpec((r, k), lambda i: (i, 0)),
            pl.BlockSpec((r, k), lambda i: (i, 0)),
        ],
        out_shape=[
            jax.ShapeDtypeStruct((b, d), jnp.float32),
            jax.ShapeDtypeStruct((b, k), jnp.int32),
            jax.ShapeDtypeStruct((b, k), jnp.float32),
        ],
        scratch_shapes=[
            pltpu.VMEM((r, _QDEPTH, d // _CHUNK), jnp.float32),
            pltpu.VMEM((r, _QDEPTH, d // _CHUNK), jnp.float32),
            pltpu.VMEM((r, _QDEPTH, d // _CHUNK), jnp.int32),
        ],
        compiler_params=pltpu.CompilerParams(
            dimension_semantics=("parallel",)),
    )(dense_code)


def _decode_body(sc_ref, w_ref, out_ref):
    kk = pl.program_id(1)

    @pl.when(kk == 0)
    def _():
        out_ref[...] = jnp.zeros_like(out_ref)

    out_ref[...] = out_ref[...] + jax.lax.dot_general(
        sc_ref[...].astype(jnp.bfloat16), w_ref[...],
        (((1,), (1,)), ((), ())),
        preferred_element_type=jnp.float32)


def _decode(sparse_code, W_dec_bf16):
    b, d = sparse_code.shape
    i_dim = W_dec_bf16.shape[0]
    r = min(1024, b)
    k_tile = min(512, d)
    grid = (b // r, d // k_tile)
    return pl.pallas_call(
        _decode_body,
        grid=grid,
        in_specs=[
            pl.BlockSpec((r, k_tile), lambda i, kk: (i, kk)),
            pl.BlockSpec((i_dim, k_tile), lambda i, kk: (0, kk)),
        ],
        out_specs=pl.BlockSpec((r, i_dim), lambda i, kk: (i, 0)),
        out_shape=jax.ShapeDtypeStruct((b, i_dim), jnp.float32),
        compiler_params=pltpu.CompilerParams(
            dimension_semantics=("parallel", "arbitrary")),
    )(sparse_code, W_dec_bf16)


def kernel(x, W_enc, b_enc, W_dec):
    x_hi = x.astype(jnp.bfloat16)
    x_lo = (x - x_hi.astype(jnp.float32)).astype(jnp.bfloat16)
    W_hi = W_enc.astype(jnp.bfloat16)
    W_lo = (W_enc - W_hi.astype(jnp.float32)).astype(jnp.bfloat16)
    dense_code = _encode_matmul(x_hi, x_lo, W_hi, W_lo, b_enc)
    sparse_code, active_indices, _vals = _topk(dense_code, TOPK)
    reconstructed_x = _decode(sparse_code, W_dec.astype(jnp.bfloat16))
    return reconstructed_x, sparse_code, active_indices


# speedup vs baseline: 1.5361x; 1.5361x over previous
import functools

import jax
import jax.numpy as jnp
from jax.experimental import pallas as pl
from jax.experimental.pallas import tpu as pltpu

TOPK = 64
_CHUNK = 128
_QDEPTH = 8


def _matmul_body(x_ref, w_ref, b_ref, out_ref):
    out_ref[...] = jax.lax.dot_general(
        x_ref[...], w_ref[...], (((1,), (1,)), ((), ())),
        preferred_element_type=jnp.float32) + b_ref[...]


def _encode_matmul(x, W_enc, b_enc):
    b, i_dim = x.shape
    d = W_enc.shape[0]
    r = min(1024, b)
    n_tile = min(512, d)
    grid = (b // r, d // n_tile)
    return pl.pallas_call(
        _matmul_body,
        grid=grid,
        in_specs=[
            pl.BlockSpec((r, i_dim), lambda i, j: (i, 0)),
            pl.BlockSpec((n_tile, i_dim), lambda i, j: (j, 0)),
            pl.BlockSpec((1, n_tile), lambda i, j: (0, j)),
        ],
        out_specs=pl.BlockSpec((r, n_tile), lambda i, j: (i, j)),
        out_shape=jax.ShapeDtypeStruct((b, d), jnp.float32),
        compiler_params=pltpu.CompilerParams(
            dimension_semantics=("parallel", "arbitrary")),
    )(x, W_enc, b_enc.reshape(1, d))


def _phase1_body(code_ref, qav_out, qsv_out, qix_out, *, r, d):
    w = _CHUNK
    nc = d // w
    q_depth = _QDEPTH
    c3 = code_ref[...].reshape(r, nc, w)
    iota_w = jax.lax.broadcasted_iota(jnp.int32, (r, nc, w), 2)

    a3 = jnp.abs(c3)
    for q in range(q_depth):
        m = jnp.max(a3, axis=2)
        sel = jnp.min(jnp.where(a3 == m[:, :, None], iota_w, w), axis=2)
        is_sel = iota_w == sel[:, :, None]
        qav_out[:, q, :] = m
        qsv_out[:, q, :] = jnp.sum(jnp.where(is_sel, c3, 0.0), axis=2)
        qix_out[:, q, :] = sel
        a3 = jnp.where(is_sel, -1.0, a3)


def _phase1(dense_code):
    b, d = dense_code.shape
    r = min(64, b)
    nc = d // _CHUNK
    body = functools.partial(_phase1_body, r=r, d=d)
    return pl.pallas_call(
        body,
        grid=(b // r,),
        in_specs=[pl.BlockSpec((r, d), lambda i: (i, 0))],
        out_specs=[
            pl.BlockSpec((r, _QDEPTH, nc), lambda i: (i, 0, 0)),
            pl.BlockSpec((r, _QDEPTH, nc), lambda i: (i, 0, 0)),
            pl.BlockSpec((r, _QDEPTH, nc), lambda i: (i, 0, 0)),
        ],
        out_shape=[
            jax.ShapeDtypeStruct((b, _QDEPTH, nc), jnp.float32),
            jax.ShapeDtypeStruct((b, _QDEPTH, nc), jnp.float32),
            jax.ShapeDtypeStruct((b, _QDEPTH, nc), jnp.int32),
        ],
        compiler_params=pltpu.CompilerParams(
            dimension_semantics=("parallel",)),
    )(dense_code)


def _phase2_body(qav_ref, qsv_ref, qix_ref, idx_out, val_out, *, k, r, nc):
    w = _CHUNK
    q_depth = _QDEPTH
    iota_q = jax.lax.broadcasted_iota(jnp.int32, (r, q_depth, nc), 1)
    iota_nc = jax.lax.broadcasted_iota(jnp.int32, (r, nc), 1)
    iota_k = jax.lax.broadcasted_iota(jnp.int32, (r, k), 1)
    qav = qav_ref[...]
    qsv = qsv_ref[...]
    qix = qix_ref[...]

    def body(t, carry):
        head, ptr, vals, idxs = carry
        m = jnp.max(head, axis=1, keepdims=True)
        c = jnp.min(jnp.where(head == m, iota_nc, nc), axis=1, keepdims=True)
        onehot = iota_nc == c
        mask3 = onehot[:, None, :] & (iota_q == ptr[:, None, :])
        v = jnp.sum(jnp.sum(jnp.where(mask3, qsv, 0.0), axis=1), axis=1,
                    keepdims=True)
        wi = jnp.sum(jnp.sum(jnp.where(mask3, qix, 0), axis=1), axis=1,
                     keepdims=True)
        gidx = c * w + wi
        ptr = ptr + onehot.astype(jnp.int32)
        head_new = jnp.max(jnp.where(iota_q == ptr[:, None, :], qav, -1.0),
                           axis=1)
        head = jnp.where(onehot, head_new, head)
        vals = jnp.where(iota_k == t, v, vals)
        idxs = jnp.where(iota_k == t, gidx, idxs)
        return head, ptr, vals, idxs

    init = (qav[:, 0, :], jnp.zeros((r, nc), jnp.int32),
            jnp.zeros((r, k), jnp.float32), jnp.zeros((r, k), jnp.int32))
    _, _, vals, idxs = jax.lax.fori_loop(0, k, body, init)
    val_out[...] = vals
    idx_out[...] = idxs


def _phase2(qav, qsv, qix, k):
    b, q_depth, nc = qav.shape
    r = min(512, b)
    body = functools.partial(_phase2_body, k=k, r=r, nc=nc)
    return pl.pallas_call(
        body,
        grid=(b // r,),
        in_specs=[
            pl.BlockSpec((r, q_depth, nc), lambda i: (i, 0, 0)),
            pl.BlockSpec((r, q_depth, nc), lambda i: (i, 0, 0)),
            pl.BlockSpec((r, q_depth, nc), lambda i: (i, 0, 0)),
        ],
        out_specs=[
            pl.BlockSpec((r, k), lambda i: (i, 0)),
            pl.BlockSpec((r, k), lambda i: (i, 0)),
        ],
        out_shape=[
            jax.ShapeDtypeStruct((b, k), jnp.int32),
            jax.ShapeDtypeStruct((b, k), jnp.float32),
        ],
        compiler_params=pltpu.CompilerParams(
            dimension_semantics=("parallel",)),
    )(qav, qsv, qix)


def _decode_body(code_ref, val_ref, w_ref, out_ref, sc_out, *, k):
    kk = pl.program_id(1)

    @pl.when(kk == 0)
    def _():
        out_ref[...] = jnp.zeros_like(out_ref)

    th = jnp.abs(val_ref[:, k - 1:k])
    code = code_ref[...]
    sc = jnp.where(jnp.abs(code) >= th, code, 0.0)
    sc_out[...] = sc
    out_ref[...] = out_ref[...] + jax.lax.dot_general(
        sc.astype(jnp.bfloat16), w_ref[...],
        (((1,), (1,)), ((), ())),
        preferred_element_type=jnp.float32)


def _decode(dense_code, vals, W_dec_bf16, k):
    b, d = dense_code.shape
    i_dim = W_dec_bf16.shape[0]
    r = min(1024, b)
    k_tile = min(512, d)
    grid = (b // r, d // k_tile)
    body = functools.partial(_decode_body, k=k)
    return pl.pallas_call(
        body,
        grid=grid,
        in_specs=[
            pl.BlockSpec((r, k_tile), lambda i, kk: (i, kk)),
            pl.BlockSpec((r, k), lambda i, kk: (i, 0)),
            pl.BlockSpec((i_dim, k_tile), lambda i, kk: (0, kk)),
        ],
        out_specs=[
            pl.BlockSpec((r, i_dim), lambda i, kk: (i, 0)),
            pl.BlockSpec((r, k_tile), lambda i, kk: (i, kk)),
        ],
        out_shape=[
            jax.ShapeDtypeStruct((b, i_dim), jnp.float32),
            jax.ShapeDtypeStruct((b, d), jnp.float32),
        ],
        compiler_params=pltpu.CompilerParams(
            dimension_semantics=("parallel", "arbitrary")),
    )(dense_code, vals, W_dec_bf16)


def kernel(x, W_enc, b_enc, W_dec):
    dense_code = _encode_matmul(x, W_enc, b_enc)
    qav, qsv, qix = _phase1(dense_code)
    active_indices, vals = _phase2(qav, qsv, qix, TOPK)
    reconstructed_x, sparse_code = _decode(
        dense_code, vals, W_dec.astype(jnp.bfloat16), TOPK)
    return reconstructed_x, sparse_code, active_indices


# speedup vs baseline: 1.5524x; 1.0106x over previous
import functools

import jax
import jax.numpy as jnp
from jax.experimental import pallas as pl
from jax.experimental.pallas import tpu as pltpu

TOPK = 64
_CHUNK = 128
_QDEPTH = 8


def _matmul_body(x_ref, w_ref, b_ref, out_ref):
    out_ref[...] = jax.lax.dot_general(
        x_ref[...], w_ref[...], (((1,), (1,)), ((), ())),
        preferred_element_type=jnp.float32) + b_ref[...]


def _encode_matmul(x, W_enc, b_enc):
    b, i_dim = x.shape
    d = W_enc.shape[0]
    r = min(1024, b)
    n_tile = min(512, d)
    grid = (b // r, d // n_tile)
    return pl.pallas_call(
        _matmul_body,
        grid=grid,
        in_specs=[
            pl.BlockSpec((r, i_dim), lambda i, j: (i, 0)),
            pl.BlockSpec((n_tile, i_dim), lambda i, j: (j, 0)),
            pl.BlockSpec((1, n_tile), lambda i, j: (0, j)),
        ],
        out_specs=pl.BlockSpec((r, n_tile), lambda i, j: (i, j)),
        out_shape=jax.ShapeDtypeStruct((b, d), jnp.float32),
        compiler_params=pltpu.CompilerParams(
            dimension_semantics=("parallel", "arbitrary")),
    )(x, W_enc, b_enc.reshape(1, d))


def _phase1_body(code_ref, qav_out, qsv_out, qix_out, *, r, d):
    w = _CHUNK
    nc = d // w
    q_depth = _QDEPTH
    c3 = code_ref[...].reshape(r, nc, w)
    iota_w = jax.lax.broadcasted_iota(jnp.int32, (r, nc, w), 2)

    a3 = jnp.abs(c3)
    for q in range(q_depth):
        m = jnp.max(a3, axis=2)
        sel = jnp.min(jnp.where(a3 == m[:, :, None], iota_w, w), axis=2)
        is_sel = iota_w == sel[:, :, None]
        qav_out[:, q, :] = m
        qsv_out[:, q, :] = jnp.sum(jnp.where(is_sel, c3, 0.0), axis=2)
        qix_out[:, q, :] = sel
        a3 = jnp.where(is_sel, -1.0, a3)


def _phase1(dense_code):
    b, d = dense_code.shape
    r = min(64, b)
    nc = d // _CHUNK
    body = functools.partial(_phase1_body, r=r, d=d)
    return pl.pallas_call(
        body,
        grid=(b // r,),
        in_specs=[pl.BlockSpec((r, d), lambda i: (i, 0))],
        out_specs=[
            pl.BlockSpec((r, _QDEPTH, nc), lambda i: (i, 0, 0)),
            pl.BlockSpec((r, _QDEPTH, nc), lambda i: (i, 0, 0)),
            pl.BlockSpec((r, _QDEPTH, nc), lambda i: (i, 0, 0)),
        ],
        out_shape=[
            jax.ShapeDtypeStruct((b, _QDEPTH, nc), jnp.float32),
            jax.ShapeDtypeStruct((b, _QDEPTH, nc), jnp.float32),
            jax.ShapeDtypeStruct((b, _QDEPTH, nc), jnp.int32),
        ],
        compiler_params=pltpu.CompilerParams(
            dimension_semantics=("parallel",)),
    )(dense_code)


def _phase2_body(qav_ref, qsv_ref, qix_ref, idx_out, val_out, *, k, r, nc):
    w = _CHUNK
    q_depth = _QDEPTH
    iota_q = jax.lax.broadcasted_iota(jnp.int32, (r, q_depth, nc), 1)
    iota_nc = jax.lax.broadcasted_iota(jnp.int32, (r, nc), 1)
    iota_k = jax.lax.broadcasted_iota(jnp.int32, (r, k), 1)
    qav = qav_ref[...]
    qsv = qsv_ref[...]
    qix = qix_ref[...]

    def body(t, carry):
        head, ptr, vals, idxs = carry
        m = jnp.max(head, axis=1, keepdims=True)
        c = jnp.min(jnp.where(head == m, iota_nc, nc), axis=1, keepdims=True)
        onehot = iota_nc == c
        mask3 = onehot[:, None, :] & (iota_q == ptr[:, None, :])
        v = jnp.sum(jnp.sum(jnp.where(mask3, qsv, 0.0), axis=1), axis=1,
                    keepdims=True)
        wi = jnp.sum(jnp.sum(jnp.where(mask3, qix, 0), axis=1), axis=1,
                     keepdims=True)
        gidx = c * w + wi
        ptr = ptr + onehot.astype(jnp.int32)
        head_new = jnp.max(jnp.where(iota_q == ptr[:, None, :], qav, -1.0),
                           axis=1)
        head = jnp.where(onehot, head_new, head)
        vals = jnp.where(iota_k == t, v, vals)
        idxs = jnp.where(iota_k == t, gidx, idxs)
        return head, ptr, vals, idxs

    init = (qav[:, 0, :], jnp.zeros((r, nc), jnp.int32),
            jnp.zeros((r, k), jnp.float32), jnp.zeros((r, k), jnp.int32))
    _, _, vals, idxs = jax.lax.fori_loop(0, k, body, init)
    val_out[...] = vals
    idx_out[...] = idxs


def _phase2(qav, qsv, qix, k):
    b, q_depth, nc = qav.shape
    r = min(1024, b)
    body = functools.partial(_phase2_body, k=k, r=r, nc=nc)
    return pl.pallas_call(
        body,
        grid=(b // r,),
        in_specs=[
            pl.BlockSpec((r, q_depth, nc), lambda i: (i, 0, 0)),
            pl.BlockSpec((r, q_depth, nc), lambda i: (i, 0, 0)),
            pl.BlockSpec((r, q_depth, nc), lambda i: (i, 0, 0)),
        ],
        out_specs=[
            pl.BlockSpec((r, k), lambda i: (i, 0)),
            pl.BlockSpec((r, k), lambda i: (i, 0)),
        ],
        out_shape=[
            jax.ShapeDtypeStruct((b, k), jnp.int32),
            jax.ShapeDtypeStruct((b, k), jnp.float32),
        ],
        compiler_params=pltpu.CompilerParams(
            dimension_semantics=("parallel",)),
    )(qav, qsv, qix)


def _decode_body(code_ref, val_ref, w_ref, out_ref, sc_out, *, k):
    kk = pl.program_id(1)

    @pl.when(kk == 0)
    def _():
        out_ref[...] = jnp.zeros_like(out_ref)

    th = jnp.abs(val_ref[:, k - 1:k])
    code = code_ref[...]
    sc = jnp.where(jnp.abs(code) >= th, code, 0.0)
    sc_out[...] = sc
    out_ref[...] = out_ref[...] + jax.lax.dot_general(
        sc.astype(jnp.bfloat16), w_ref[...],
        (((1,), (1,)), ((), ())),
        preferred_element_type=jnp.float32)


def _decode(dense_code, vals, W_dec_bf16, k):
    b, d = dense_code.shape
    i_dim = W_dec_bf16.shape[0]
    r = min(1024, b)
    k_tile = min(512, d)
    grid = (b // r, d // k_tile)
    body = functools.partial(_decode_body, k=k)
    return pl.pallas_call(
        body,
        grid=grid,
        in_specs=[
            pl.BlockSpec((r, k_tile), lambda i, kk: (i, kk)),
            pl.BlockSpec((r, k), lambda i, kk: (i, 0)),
            pl.BlockSpec((i_dim, k_tile), lambda i, kk: (0, kk)),
        ],
        out_specs=[
            pl.BlockSpec((r, i_dim), lambda i, kk: (i, 0)),
            pl.BlockSpec((r, k_tile), lambda i, kk: (i, kk)),
        ],
        out_shape=[
            jax.ShapeDtypeStruct((b, i_dim), jnp.float32),
            jax.ShapeDtypeStruct((b, d), jnp.float32),
        ],
        compiler_params=pltpu.CompilerParams(
            dimension_semantics=("parallel", "arbitrary")),
    )(dense_code, vals, W_dec_bf16)


def kernel(x, W_enc, b_enc, W_dec):
    dense_code = _encode_matmul(x, W_enc, b_enc)
    qav, qsv, qix = _phase1(dense_code)
    active_indices, vals = _phase2(qav, qsv, qix, TOPK)
    reconstructed_x, sparse_code = _decode(
        dense_code, vals, W_dec.astype(jnp.bfloat16), TOPK)
    return reconstructed_x, sparse_code, active_indices


# speedup vs baseline: 1.8738x; 1.2070x over previous
import functools

import jax
import jax.numpy as jnp
from jax.experimental import pallas as pl
from jax.experimental.pallas import tpu as pltpu

TOPK = 64
_CHUNK = 128
_QDEPTH = 8


def _matmul_body(x_ref, w_ref, b_ref, out_ref):
    out_ref[...] = jax.lax.dot_general(
        x_ref[...], w_ref[...], (((1,), (1,)), ((), ())),
        preferred_element_type=jnp.float32) + b_ref[...]


def _encode_matmul(x, W_enc, b_enc):
    b, i_dim = x.shape
    d = W_enc.shape[0]
    r = min(1024, b)
    n_tile = min(512, d)
    grid = (b // r, d // n_tile)
    return pl.pallas_call(
        _matmul_body,
        grid=grid,
        in_specs=[
            pl.BlockSpec((r, i_dim), lambda i, j: (i, 0)),
            pl.BlockSpec((n_tile, i_dim), lambda i, j: (j, 0)),
            pl.BlockSpec((1, n_tile), lambda i, j: (0, j)),
        ],
        out_specs=pl.BlockSpec((r, n_tile), lambda i, j: (i, j)),
        out_shape=jax.ShapeDtypeStruct((b, d), jnp.float32),
        compiler_params=pltpu.CompilerParams(
            dimension_semantics=("parallel", "arbitrary")),
    )(x, W_enc, b_enc.reshape(1, d))


def _phase1_body(code_ref, qav_out, qpk_out, *, r, d):
    w = _CHUNK
    nc = d // w
    q_depth = _QDEPTH
    c3 = code_ref[...].reshape(r, nc, w)
    iota_w = jax.lax.broadcasted_iota(jnp.int32, (r, nc, w), 2)
    iota_s = 2 * iota_w + jnp.where(c3 < 0.0, 1, 0)

    a3 = jnp.abs(c3)
    for q in range(q_depth):
        m = jnp.max(a3, axis=2)
        sel = jnp.min(jnp.where(a3 == m[:, :, None], iota_s, 2 * w), axis=2)
        is_sel = iota_s == sel[:, :, None]
        qav_out[:, q, :] = m
        qpk_out[:, q, :] = sel
        a3 = jnp.where(is_sel, -1.0, a3)


def _phase1(dense_code):
    b, d = dense_code.shape
    r = min(64, b)
    nc = d // _CHUNK
    body = functools.partial(_phase1_body, r=r, d=d)
    return pl.pallas_call(
        body,
        grid=(b // r,),
        in_specs=[pl.BlockSpec((r, d), lambda i: (i, 0))],
        out_specs=[
            pl.BlockSpec((r, _QDEPTH, nc), lambda i: (i, 0, 0)),
            pl.BlockSpec((r, _QDEPTH, nc), lambda i: (i, 0, 0)),
        ],
        out_shape=[
            jax.ShapeDtypeStruct((b, _QDEPTH, nc), jnp.float32),
            jax.ShapeDtypeStruct((b, _QDEPTH, nc), jnp.int32),
        ],
        compiler_params=pltpu.CompilerParams(
            dimension_semantics=("parallel",)),
    )(dense_code)


def _phase2_body(qav_ref, qpk_ref, idx_out, val_out, *, k, r, nc):
    w = _CHUNK
    q_depth = _QDEPTH
    iota_q = jax.lax.broadcasted_iota(jnp.int32, (r, q_depth, nc), 1)
    iota_nc = jax.lax.broadcasted_iota(jnp.int32, (r, nc), 1)
    iota_k = jax.lax.broadcasted_iota(jnp.int32, (r, k), 1)
    qav = qav_ref[...]
    qpk = qpk_ref[...]

    def body(t, carry):
        head_av, head_pk, ptr, vals, idxs = carry
        m = jnp.max(head_av, axis=1, keepdims=True)
        c = jnp.min(jnp.where(head_av == m, iota_nc, nc), axis=1,
                    keepdims=True)
        onehot = iota_nc == c
        pk = jnp.max(jnp.where(onehot, head_pk, -1), axis=1, keepdims=True)
        v = jnp.where(pk % 2 == 1, -m, m)
        gidx = c * w + pk // 2
        ptr = ptr + onehot.astype(jnp.int32)
        gmask = iota_q == ptr[:, None, :]
        av_new = jnp.max(jnp.where(gmask, qav, -1.0), axis=1)
        pk_new = jnp.max(jnp.where(gmask, qpk, -1), axis=1)
        head_av = jnp.where(onehot, av_new, head_av)
        head_pk = jnp.where(onehot, pk_new, head_pk)
        vals = jnp.where(iota_k == t, v, vals)
        idxs = jnp.where(iota_k == t, gidx, idxs)
        return head_av, head_pk, ptr, vals, idxs

    init = (qav[:, 0, :], qpk[:, 0, :], jnp.zeros((r, nc), jnp.int32),
            jnp.zeros((r, k), jnp.float32), jnp.zeros((r, k), jnp.int32))
    _, _, _, vals, idxs = jax.lax.fori_loop(0, k, body, init)
    val_out[...] = vals
    idx_out[...] = idxs


def _phase2(qav, qpk, k):
    b, q_depth, nc = qav.shape
    r = min(1024, b)
    body = functools.partial(_phase2_body, k=k, r=r, nc=nc)
    return pl.pallas_call(
        body,
        grid=(b // r,),
        in_specs=[
            pl.BlockSpec((r, q_depth, nc), lambda i: (i, 0, 0)),
            pl.BlockSpec((r, q_depth, nc), lambda i: (i, 0, 0)),
        ],
        out_specs=[
            pl.BlockSpec((r, k), lambda i: (i, 0)),
            pl.BlockSpec((r, k), lambda i: (i, 0)),
        ],
        out_shape=[
            jax.ShapeDtypeStruct((b, k), jnp.int32),
            jax.ShapeDtypeStruct((b, k), jnp.float32),
        ],
        compiler_params=pltpu.CompilerParams(
            dimension_semantics=("parallel",)),
    )(qav, qpk)


def _decode_body(code_ref, val_ref, w_ref, out_ref, sc_out, *, k):
    kk = pl.program_id(1)

    @pl.when(kk == 0)
    def _():
        out_ref[...] = jnp.zeros_like(out_ref)

    th = jnp.abs(val_ref[:, k - 1:k])
    code = code_ref[...]
    sc = jnp.where(jnp.abs(code) >= th, code, 0.0)
    sc_out[...] = sc
    out_ref[...] = out_ref[...] + jax.lax.dot_general(
        sc.astype(jnp.bfloat16), w_ref[...],
        (((1,), (1,)), ((), ())),
        preferred_element_type=jnp.float32)


def _decode(dense_code, vals, W_dec_bf16, k):
    b, d = dense_code.shape
    i_dim = W_dec_bf16.shape[0]
    r = min(1024, b)
    k_tile = min(512, d)
    grid = (b // r, d // k_tile)
    body = functools.partial(_decode_body, k=k)
    return pl.pallas_call(
        body,
        grid=grid,
        in_specs=[
            pl.BlockSpec((r, k_tile), lambda i, kk: (i, kk)),
            pl.BlockSpec((r, k), lambda i, kk: (i, 0)),
            pl.BlockSpec((i_dim, k_tile), lambda i, kk: (0, kk)),
        ],
        out_specs=[
            pl.BlockSpec((r, i_dim), lambda i, kk: (i, 0)),
            pl.BlockSpec((r, k_tile), lambda i, kk: (i, kk)),
        ],
        out_shape=[
            jax.ShapeDtypeStruct((b, i_dim), jnp.float32),
            jax.ShapeDtypeStruct((b, d), jnp.float32),
        ],
        compiler_params=pltpu.CompilerParams(
            dimension_semantics=("parallel", "arbitrary")),
    )(dense_code, vals, W_dec_bf16)


def kernel(x, W_enc, b_enc, W_dec):
    dense_code = _encode_matmul(x, W_enc, b_enc)
    qav, qpk = _phase1(dense_code)
    active_indices, vals = _phase2(qav, qpk, TOPK)
    reconstructed_x, sparse_code = _decode(
        dense_code, vals, W_dec.astype(jnp.bfloat16), TOPK)
    return reconstructed_x, sparse_code, active_indices
